# baseline (speedup 1.0000x reference)
import functools

import jax
import jax.numpy as jnp
from jax import lax
from jax.experimental import pallas as pl
from jax.experimental.pallas import tpu as pltpu
from jax.experimental.pallas import tpu_sc as plsc


def _ln(h, scale, bias):
    m = jnp.mean(h, axis=-1, keepdims=True)
    v = jnp.mean((h - m) * (h - m), axis=-1, keepdims=True)
    return (h - m) * lax.rsqrt(v + 1e-5) * scale + bias


def _preproj_body(mesh_ref, grid_ref, w1s_ref, w1d_ref, b1_ref, ps_ref, pd_ref):
    ps_ref[...] = jnp.dot(mesh_ref[...], w1s_ref[...],
                          preferred_element_type=jnp.float32)
    pd_ref[...] = jnp.dot(grid_ref[...], w1d_ref[...],
                          preferred_element_type=jnp.float32) + b1_ref[...]


def _tc_preproj(mesh_f, grid_f, w1s, w1d, b1):
    n, h = mesh_f.shape
    tn = 1000
    rows = pl.BlockSpec((tn, h), lambda i: (i, 0))
    full = pl.BlockSpec((h, h), lambda i: (0, 0))
    vec = pl.BlockSpec((1, h), lambda i: (0, 0))
    return pl.pallas_call(
        _preproj_body,
        grid=(n // tn,),
        in_specs=[rows, rows, full, full, vec],
        out_specs=[rows, rows],
        out_shape=[jax.ShapeDtypeStruct((n, h), jnp.float32)] * 2,
    )(mesh_f, grid_f, w1s, w1d, b1.reshape(1, h))


def _edge_body(e_ref, ga_ref, gb_ref, w1e_ref, w2_ref, b2_ref, s_ref, b_ref,
               out_ref):
    e = e_ref[...]
    h = jnp.dot(e.astype(jnp.bfloat16), w1e_ref[...],
                preferred_element_type=jnp.float32)
    h = h + ga_ref[...] + gb_ref[...]
    h = h * jax.nn.sigmoid(h)
    h = jnp.dot(h.astype(jnp.bfloat16), w2_ref[...],
                preferred_element_type=jnp.float32) + b2_ref[...]
    ep = e + _ln(h, s_ref[...], b_ref[...])
    lw = 128
    for k in range(4):
        out_ref[k] = ep[:, k * lw:(k + 1) * lw]


def _tc_edge(e, ga, gb, w1e, w2, b2, sc, bi):
    ne, h = e.shape
    te = next(x for x in (1280, 800, 640, 400, 160) if ne % x == 0)
    rows = pl.BlockSpec((te, h), lambda i: (i, 0))
    full = pl.BlockSpec((h, h), lambda i: (0, 0))
    vec = pl.BlockSpec((1, h), lambda i: (0, 0))
    blocked = pl.BlockSpec((4, te, 128), lambda i: (0, i, 0))
    return pl.pallas_call(
        _edge_body,
        grid=(ne // te,),
        in_specs=[rows, rows, rows, full, full, vec, vec, vec],
        out_specs=blocked,
        out_shape=jax.ShapeDtypeStruct((4, ne, 128), jnp.float32),
    )(e, ga, gb, w1e.astype(jnp.bfloat16), w2.astype(jnp.bfloat16),
      b2.reshape(1, h), sc.reshape(1, h), bi.reshape(1, h))


def _node_body(n_ref, agg1_ref, agg2_ref, w1a_ref, w1b_ref, b1_ref, w2_ref,
               b2_ref, s_ref, b_ref, out_ref):
    nd = n_ref[...]
    h = jnp.dot(nd, w1a_ref[...], preferred_element_type=jnp.float32)
    h = h + jnp.dot(agg1_ref[...] + agg2_ref[...], w1b_ref[...],
                    preferred_element_type=jnp.float32) + b1_ref[...]
    h = h * jax.nn.sigmoid(h)
    h = jnp.dot(h, w2_ref[...], preferred_element_type=jnp.float32) + b2_ref[...]
    out_ref[...] = nd + _ln(h, s_ref[...], b_ref[...])


def _tc_node(node_f, agg1, agg2, w1a, w1b, b1, w2, b2, sc, bi):
    n, h = node_f.shape
    tn = 1000
    rows = pl.BlockSpec((tn, h), lambda i: (i, 0))
    full = pl.BlockSpec((h, h), lambda i: (0, 0))
    vec = pl.BlockSpec((1, h), lambda i: (0, 0))
    return pl.pallas_call(
        _node_body,
        grid=(n // tn,),
        in_specs=[rows, rows, rows, full, full, vec, full, vec, vec, vec],
        out_specs=rows,
        out_shape=jax.ShapeDtypeStruct((n, h), jnp.float32),
    )(node_f, agg1, agg2, w1a, w1b, b1.reshape(1, h), w2, b2.reshape(1, h),
      sc.reshape(1, h), bi.reshape(1, h))


def _sc_gather(psrc, pdst, src_idx, dst_idx):
    n, h = psrc.shape
    e = src_idx.shape[0]
    nw = 32
    per_w = e // nw
    c = 40
    nch = per_w // c
    src3 = src_idx.reshape(nw, nch, c)
    dst3 = dst_idx.reshape(nw, nch, c)
    mesh = plsc.VectorSubcoreMesh(core_axis_name="c", subcore_axis_name="s")

    bufty = pltpu.VMEM((c, h), jnp.float32)

    @functools.partial(
        pl.kernel, mesh=mesh,
        out_type=[jax.ShapeDtypeStruct((e, h), jnp.float32),
                  jax.ShapeDtypeStruct((e, h), jnp.float32)],
        scratch_types=[
            pltpu.VMEM((nch, c), jnp.int32), pltpu.VMEM((nch, c), jnp.int32),
            bufty, bufty, bufty, bufty,
            pltpu.SemaphoreType.DMA, pltpu.SemaphoreType.DMA,
            pltpu.SemaphoreType.DMA, pltpu.SemaphoreType.DMA,
        ])
    def k(ps_hbm, pd_hbm, si_hbm, di_hbm, ga_hbm, gb_hbm,
          idxs, idxd, bufa0, bufa1, bufb0, bufb1, semg0, semg1, semw0, semw1):
        wid = lax.axis_index("s") * 2 + lax.axis_index("c")
        pltpu.sync_copy(si_hbm.at[wid], idxs)
        pltpu.sync_copy(di_hbm.at[wid], idxd)
        bufs = ((bufa0, bufb0, semg0, semw0), (bufa1, bufb1, semg1, semw1))

        def gath(j, p):
            ba, bb, sg, _ = bufs[p]
            pltpu.async_copy(ps_hbm.at[idxs.at[j]], ba, sg)
            pltpu.async_copy(pd_hbm.at[idxd.at[j]], bb, sg)

        def waitg(p):
            ba, bb, sg, _ = bufs[p]
            pltpu.make_async_copy(ps_hbm.at[idxs.at[0]], ba, sg).wait()
            pltpu.make_async_copy(pd_hbm.at[idxd.at[0]], bb, sg).wait()

        def wrote(j, p):
            ba, bb, _, sw = bufs[p]
            base = wid * per_w + j * c
            pltpu.async_copy(ba, ga_hbm.at[pl.ds(base, c)], sw)
            pltpu.async_copy(bb, gb_hbm.at[pl.ds(base, c)], sw)

        def waitw(p):
            ba, bb, _, sw = bufs[p]
            pltpu.make_async_copy(ba, ga_hbm.at[pl.ds(0, c)], sw).wait()
            pltpu.make_async_copy(bb, gb_hbm.at[pl.ds(0, c)], sw).wait()

        gath(0, 0)
        gath(1, 1)

        def body(i, carry):
            j0 = 2 * i
            waitg(0)
            wrote(j0, 0)
            waitg(1)
            wrote(j0 + 1, 1)
            waitw(0)

            @pl.when(j0 + 2 < nch)
            def _():
                gath(j0 + 2, 0)

            waitw(1)

            @pl.when(j0 + 3 < nch)
            def _():
                gath(j0 + 3, 1)

            return carry

        lax.fori_loop(0, nch // 2, body, 0)
        if nch % 2:
            waitg(0)
            wrote(nch - 1, 0)
            waitw(0)

    return k(psrc, pdst, src3, dst3)


def _sc_scatter(eprime, dst_idx, zrows):
    nblk4, e, cb = eprime.shape
    n = zrows.shape[0]
    nblk = nblk4 // 2
    per_tile = e // 16
    cs = 80
    nch = per_tile // cs
    rc = 400
    nrc = n // rc
    nrit = (nrc + 15) // 16
    dst4 = dst_idx.reshape(16, nch, 1, cs)
    mesh = plsc.VectorSubcoreMesh(core_axis_name="c", subcore_axis_name="s")

    @functools.partial(
        pl.kernel, mesh=mesh,
        out_type=jax.ShapeDtypeStruct((n, nblk4 * cb), jnp.float32),
        scratch_types=[
            pltpu.VMEM((nch, 1, cs), jnp.int32),
            pltpu.VMEM((cs, cb), jnp.float32),
            pltpu.VMEM((cs, cb), jnp.float32),
            pltpu.VMEM_SHARED((n, cb), jnp.float32),
            pltpu.SemaphoreType.DMA, pltpu.SemaphoreType.DMA,
        ])
    def k(ep_hbm, di_hbm, z_hbm, agg_hbm, idxall, ebuf0, ebuf1, acc,
          sem0, sem1):
        c = lax.axis_index("c")
        s = lax.axis_index("s")
        pltpu.sync_copy(di_hbm.at[s], idxall)
        bufs = ((ebuf0, sem0), (ebuf1, sem1))

        for bb in range(nblk):
            blk = bb * 2 + c
            col = blk * cb

            def zbody(j, carry):
                idx = j * 16 + s

                @pl.when(idx < nrc)
                def _():
                    rb = idx * rc
                    pltpu.sync_copy(z_hbm.at[pl.ds(rb, rc)],
                                    acc.at[pl.ds(rb, rc)])
                return carry

            lax.fori_loop(0, nrit, zbody, 0)
            plsc.subcore_barrier()

            def load(j, p):
                buf, sem = bufs[p]
                pltpu.async_copy(
                    ep_hbm.at[blk, pl.ds(s * per_tile + j * cs, cs)],
                    buf, sem)

            def waitl(p):
                buf, sem = bufs[p]
                pltpu.make_async_copy(
                    ep_hbm.at[blk, pl.ds(0, cs)], buf, sem).wait()

            def scat(j, p):
                buf, _ = bufs[p]
                pltpu.sync_copy(buf, acc.at[idxall.at[j, 0]], add=True)

            load(0, 0)

            def body(i, carry):
                j0 = 2 * i
                load(j0 + 1, 1)
                waitl(0)
                scat(j0, 0)
                load(j0 + 2, 0)
                waitl(1)
                scat(j0 + 1, 1)
                return carry

            npairs = (nch - 1) // 2
            lax.fori_loop(0, npairs, body, 0)
            waitl(0)
            if nch % 2 == 0:
                load(nch - 1, 1)
            scat(2 * npairs, 0)
            if nch % 2 == 0:
                waitl(1)
                scat(nch - 1, 1)
            plsc.subcore_barrier()

            def wbody(j, carry):
                idx = j * 16 + s

                @pl.when(idx < nrc)
                def _():
                    rb = idx * rc
                    pltpu.sync_copy(acc.at[pl.ds(rb, rc)],
                                    agg_hbm.at[pl.ds(rb, rc), pl.ds(col, cb)])
                return carry

            lax.fori_loop(0, nrit, wbody, 0)
            plsc.subcore_barrier()

    assert nch >= 2 and nch * cs == per_tile, (nch, per_tile)
    return k(eprime, dst4, zrows)


def kernel(mesh2grid_edge_features, grid_node_features, mesh_node_features,
           edge_index, We1, be1, We2, be2, ge_scale, ge_bias,
           Wn1, bn1, Wn2, bn2, gn_scale, gn_bias):
    h = mesh2grid_edge_features.shape[1]
    n = grid_node_features.shape[0]
    dst = edge_index[:, 0].astype(jnp.int32)
    src = edge_index[:, 1].astype(jnp.int32)

    w1e, w1s, w1d = We1[:h], We1[h:2 * h], We1[2 * h:]
    ps, pd = _tc_preproj(mesh_node_features, grid_node_features, w1s, w1d, be1)
    zrows = jnp.zeros((n, 128), jnp.float32)
    e_all = mesh2grid_edge_features
    e_cnt = e_all.shape[0]
    h1 = (e_cnt // 2) // 1280 * 1280
    aggs = []
    for lo, hi in ((0, h1), (h1, e_cnt)):
        ga, gb = _sc_gather(ps, pd, src[lo:hi], dst[lo:hi])
        ep = _tc_edge(e_all[lo:hi], ga, gb, w1e, We2, be2, ge_scale, ge_bias)
        aggs.append(_sc_scatter(ep, dst[lo:hi], zrows))
    return _tc_node(grid_node_features, aggs[0], aggs[1], Wn1[:h], Wn1[h:],
                    bn1, Wn2, bn2, gn_scale, gn_bias)


# speedup vs baseline: 1.2214x; 1.2214x over previous
import functools

import jax
import jax.numpy as jnp
from jax import lax
from jax.experimental import pallas as pl
from jax.experimental.pallas import tpu as pltpu
from jax.experimental.pallas import tpu_sc as plsc


def _ln(h, scale, bias):
    m = jnp.mean(h, axis=-1, keepdims=True)
    v = jnp.mean((h - m) * (h - m), axis=-1, keepdims=True)
    return (h - m) * lax.rsqrt(v + 1e-5) * scale + bias


def _preproj_body(mesh_ref, grid_ref, w1s_ref, w1d_ref, b1_ref, ps_ref, pd_ref):
    ps_ref[...] = jnp.dot(mesh_ref[...], w1s_ref[...],
                          preferred_element_type=jnp.float32)
    pd_ref[...] = jnp.dot(grid_ref[...], w1d_ref[...],
                          preferred_element_type=jnp.float32) + b1_ref[...]


def _tc_preproj(mesh_f, grid_f, w1s, w1d, b1):
    n, h = mesh_f.shape
    tn = 1000
    rows = pl.BlockSpec((tn, h), lambda i: (i, 0))
    full = pl.BlockSpec((h, h), lambda i: (0, 0))
    vec = pl.BlockSpec((1, h), lambda i: (0, 0))
    return pl.pallas_call(
        _preproj_body,
        grid=(n // tn,),
        in_specs=[rows, rows, full, full, vec],
        out_specs=[rows, rows],
        out_shape=[jax.ShapeDtypeStruct((n, h), jnp.float32)] * 2,
    )(mesh_f, grid_f, w1s, w1d, b1.reshape(1, h))


def _edge_body(e_ref, g_ref, w1e_ref, w2_ref, b2_ref, s_ref, b_ref,
               out_ref):
    e = e_ref[...]
    h = jnp.dot(e.astype(jnp.bfloat16), w1e_ref[...],
                preferred_element_type=jnp.float32)
    h = h + g_ref[...]
    h = h * jax.nn.sigmoid(h)
    h = jnp.dot(h.astype(jnp.bfloat16), w2_ref[...],
                preferred_element_type=jnp.float32) + b2_ref[...]
    ep = e + _ln(h, s_ref[...], b_ref[...])
    lw = 128
    for k in range(4):
        out_ref[k] = ep[:, k * lw:(k + 1) * lw]


def _tc_edge(e, g, w1e, w2, b2, sc, bi):
    ne, h = e.shape
    te = next(x for x in (1280, 800, 640, 400, 160) if ne % x == 0)
    rows = pl.BlockSpec((te, h), lambda i: (i, 0))
    full = pl.BlockSpec((h, h), lambda i: (0, 0))
    vec = pl.BlockSpec((1, h), lambda i: (0, 0))
    blocked = pl.BlockSpec((4, te, 128), lambda i: (0, i, 0))
    return pl.pallas_call(
        _edge_body,
        grid=(ne // te,),
        in_specs=[rows, rows, full, full, vec, vec, vec],
        out_specs=blocked,
        out_shape=jax.ShapeDtypeStruct((4, ne, 128), jnp.float32),
    )(e, g, w1e.astype(jnp.bfloat16), w2.astype(jnp.bfloat16),
      b2.reshape(1, h), sc.reshape(1, h), bi.reshape(1, h))


def _node_body(n_ref, agg_ref, w1a_ref, w1b_ref, b1_ref, w2_ref,
               b2_ref, s_ref, b_ref, out_ref):
    nd = n_ref[...]
    h = jnp.dot(nd, w1a_ref[...], preferred_element_type=jnp.float32)
    h = h + jnp.dot(agg_ref[...], w1b_ref[...],
                    preferred_element_type=jnp.float32) + b1_ref[...]
    h = h * jax.nn.sigmoid(h)
    h = jnp.dot(h, w2_ref[...], preferred_element_type=jnp.float32) + b2_ref[...]
    out_ref[...] = nd + _ln(h, s_ref[...], b_ref[...])


def _tc_node(node_f, agg, w1a, w1b, b1, w2, b2, sc, bi):
    n, h = node_f.shape
    tn = 1000
    rows = pl.BlockSpec((tn, h), lambda i: (i, 0))
    full = pl.BlockSpec((h, h), lambda i: (0, 0))
    vec = pl.BlockSpec((1, h), lambda i: (0, 0))
    return pl.pallas_call(
        _node_body,
        grid=(n // tn,),
        in_specs=[rows, rows, full, full, vec, full, vec, vec, vec],
        out_specs=rows,
        out_shape=jax.ShapeDtypeStruct((n, h), jnp.float32),
    )(node_f, agg, w1a, w1b, b1.reshape(1, h), w2, b2.reshape(1, h),
      sc.reshape(1, h), bi.reshape(1, h))


def _sc_gather(psrc, pdst, src_idx, dst_idx):
    n, h = psrc.shape
    e = src_idx.shape[0]
    nw = 32
    per_w = e // nw
    c = 40
    nch = per_w // c
    src3 = src_idx.reshape(nw, nch, c)
    dst3 = dst_idx.reshape(nw, nch, c)
    mesh = plsc.VectorSubcoreMesh(core_axis_name="c", subcore_axis_name="s")

    bufty = pltpu.VMEM((c, h), jnp.float32)

    @functools.partial(
        pl.kernel, mesh=mesh,
        out_type=jax.ShapeDtypeStruct((e, h), jnp.float32),
        scratch_types=[
            pltpu.VMEM((nch, c), jnp.int32), pltpu.VMEM((nch, c), jnp.int32),
            bufty, bufty, bufty, bufty,
            pltpu.SemaphoreType.DMA, pltpu.SemaphoreType.DMA,
            pltpu.SemaphoreType.DMA, pltpu.SemaphoreType.DMA,
        ])
    def k(ps_hbm, pd_hbm, si_hbm, di_hbm, gs_hbm,
          idxs, idxd, bufa0, bufa1, bufb0, bufb1, semg0, semg1, semw0, semw1):
        wid = lax.axis_index("s") * 2 + lax.axis_index("c")
        pltpu.sync_copy(si_hbm.at[wid], idxs)
        pltpu.sync_copy(di_hbm.at[wid], idxd)
        bufs = ((bufa0, bufb0, semg0, semw0), (bufa1, bufb1, semg1, semw1))

        def gath(j, p):
            ba, bb, sg, _ = bufs[p]
            pltpu.async_copy(ps_hbm.at[idxs.at[j]], ba, sg)
            pltpu.async_copy(pd_hbm.at[idxd.at[j]], bb, sg)

        def waitg(p):
            ba, bb, sg, _ = bufs[p]
            pltpu.make_async_copy(ps_hbm.at[idxs.at[0]], ba, sg).wait()
            pltpu.make_async_copy(pd_hbm.at[idxd.at[0]], bb, sg).wait()

        def addpair(p):
            ba, bb, _, _ = bufs[p]

            def arow(r, carry):
                for kk in range(h // 16):
                    sl = pl.ds(kk * 16, 16)
                    ba[r, sl] = ba[r, sl] + bb[r, sl]
                return carry

            lax.fori_loop(0, c, arow, 0)

        def wrote(j, p):
            ba, _, _, sw = bufs[p]
            base = wid * per_w + j * c
            pltpu.async_copy(ba, gs_hbm.at[pl.ds(base, c)], sw)

        def waitw(p):
            ba, _, _, sw = bufs[p]
            pltpu.make_async_copy(ba, gs_hbm.at[pl.ds(0, c)], sw).wait()

        gath(0, 0)
        gath(1, 1)

        def body(i, carry):
            j0 = 2 * i
            waitg(0)
            addpair(0)
            wrote(j0, 0)
            waitg(1)
            addpair(1)
            wrote(j0 + 1, 1)
            waitw(0)

            @pl.when(j0 + 2 < nch)
            def _():
                gath(j0 + 2, 0)

            waitw(1)

            @pl.when(j0 + 3 < nch)
            def _():
                gath(j0 + 3, 1)

            return carry

        lax.fori_loop(0, nch // 2, body, 0)
        if nch % 2:
            waitg(0)
            addpair(0)
            wrote(nch - 1, 0)
            waitw(0)

    return k(psrc, pdst, src3, dst3)


def _sc_scatter(eprime, dst_idx, zrows):
    nblk4, e, cb = eprime.shape
    n = zrows.shape[0]
    nblk = nblk4 // 2
    per_tile = e // 16
    cs = 80
    nch = per_tile // cs
    rc = 400
    nrc = n // rc
    nrit = (nrc + 15) // 16
    dst4 = dst_idx.reshape(16, nch, 1, cs)
    mesh = plsc.VectorSubcoreMesh(core_axis_name="c", subcore_axis_name="s")

    @functools.partial(
        pl.kernel, mesh=mesh,
        out_type=jax.ShapeDtypeStruct((n, nblk4 * cb), jnp.float32),
        scratch_types=[
            pltpu.VMEM((nch, 1, cs), jnp.int32),
            pltpu.VMEM((cs, cb), jnp.float32),
            pltpu.VMEM((cs, cb), jnp.float32),
            pltpu.VMEM_SHARED((n, cb), jnp.float32),
            pltpu.SemaphoreType.DMA, pltpu.SemaphoreType.DMA,
        ])
    def k(ep_hbm, di_hbm, z_hbm, agg_hbm, idxall, ebuf0, ebuf1, acc,
          sem0, sem1):
        c = lax.axis_index("c")
        s = lax.axis_index("s")
        pltpu.sync_copy(di_hbm.at[s], idxall)
        bufs = ((ebuf0, sem0), (ebuf1, sem1))

        for bb in range(nblk):
            blk = bb * 2 + c
            col = blk * cb

            def zbody(j, carry):
                idx = j * 16 + s

                @pl.when(idx < nrc)
                def _():
                    rb = idx * rc
                    pltpu.sync_copy(z_hbm.at[pl.ds(rb, rc)],
                                    acc.at[pl.ds(rb, rc)])
                return carry

            lax.fori_loop(0, nrit, zbody, 0)
            plsc.subcore_barrier()

            def load(j, p):
                buf, sem = bufs[p]
                pltpu.async_copy(
                    ep_hbm.at[blk, pl.ds(s * per_tile + j * cs, cs)],
                    buf, sem)

            def waitl(p):
                buf, sem = bufs[p]
                pltpu.make_async_copy(
                    ep_hbm.at[blk, pl.ds(0, cs)], buf, sem).wait()

            def scat(j, p):
                buf, _ = bufs[p]
                pltpu.sync_copy(buf, acc.at[idxall.at[j, 0]], add=True)

            load(0, 0)

            def body(i, carry):
                j0 = 2 * i
                load(j0 + 1, 1)
                waitl(0)
                scat(j0, 0)
                load(j0 + 2, 0)
                waitl(1)
                scat(j0 + 1, 1)
                return carry

            npairs = (nch - 1) // 2
            lax.fori_loop(0, npairs, body, 0)
            waitl(0)
            if nch % 2 == 0:
                load(nch - 1, 1)
            scat(2 * npairs, 0)
            if nch % 2 == 0:
                waitl(1)
                scat(nch - 1, 1)
            plsc.subcore_barrier()

            def wbody(j, carry):
                idx = j * 16 + s

                @pl.when(idx < nrc)
                def _():
                    rb = idx * rc
                    pltpu.sync_copy(acc.at[pl.ds(rb, rc)],
                                    agg_hbm.at[pl.ds(rb, rc), pl.ds(col, cb)])
                return carry

            lax.fori_loop(0, nrit, wbody, 0)
            plsc.subcore_barrier()

    assert nch >= 2 and nch * cs == per_tile, (nch, per_tile)
    return k(eprime, dst4, zrows)


def kernel(mesh2grid_edge_features, grid_node_features, mesh_node_features,
           edge_index, We1, be1, We2, be2, ge_scale, ge_bias,
           Wn1, bn1, Wn2, bn2, gn_scale, gn_bias):
    h = mesh2grid_edge_features.shape[1]
    n = grid_node_features.shape[0]
    dst = edge_index[:, 0].astype(jnp.int32)
    src = edge_index[:, 1].astype(jnp.int32)

    w1e, w1s, w1d = We1[:h], We1[h:2 * h], We1[2 * h:]
    ps, pd = _tc_preproj(mesh_node_features, grid_node_features, w1s, w1d, be1)
    zrows = jnp.zeros((n, 128), jnp.float32)
    gsum = _sc_gather(ps, pd, src, dst)
    ep = _tc_edge(mesh2grid_edge_features, gsum, w1e, We2, be2,
                  ge_scale, ge_bias)
    agg = _sc_scatter(ep, dst, zrows)
    return _tc_node(grid_node_features, agg, Wn1[:h], Wn1[h:],
                    bn1, Wn2, bn2, gn_scale, gn_bias)


# speedup vs baseline: 1.2280x; 1.0054x over previous
import functools

import jax
import jax.numpy as jnp
from jax import lax
from jax.experimental import pallas as pl
from jax.experimental.pallas import tpu as pltpu
from jax.experimental.pallas import tpu_sc as plsc


def _ln(h, scale, bias):
    m = jnp.mean(h, axis=-1, keepdims=True)
    v = jnp.mean(h * h, axis=-1, keepdims=True) - m * m
    k = lax.rsqrt(v + 1e-5)
    return h * (k * scale) + (bias - m * k * scale)


def _preproj_body(mesh_ref, grid_ref, w1s_ref, w1d_ref, b1_ref, ps_ref, pd_ref):
    ps_ref[...] = jnp.dot(mesh_ref[...], w1s_ref[...],
                          preferred_element_type=jnp.float32)
    pd_ref[...] = jnp.dot(grid_ref[...], w1d_ref[...],
                          preferred_element_type=jnp.float32) + b1_ref[...]


def _tc_preproj(mesh_f, grid_f, w1s, w1d, b1):
    n, h = mesh_f.shape
    tn = 1000
    rows = pl.BlockSpec((tn, h), lambda i: (i, 0))
    full = pl.BlockSpec((h, h), lambda i: (0, 0))
    vec = pl.BlockSpec((1, h), lambda i: (0, 0))
    return pl.pallas_call(
        _preproj_body,
        grid=(n // tn,),
        in_specs=[rows, rows, full, full, vec],
        out_specs=[rows, rows],
        out_shape=[jax.ShapeDtypeStruct((n, h), jnp.float32)] * 2,
    )(mesh_f, grid_f, w1s, w1d, b1.reshape(1, h))


def _edge_body(e_ref, g_ref, w1e_ref, w2_ref, b2_ref, s_ref, b_ref,
               out_ref):
    e = e_ref[...]
    h = jnp.dot(e.astype(jnp.bfloat16), w1e_ref[...],
                preferred_element_type=jnp.float32)
    h = h + g_ref[...]
    h = h * jax.nn.sigmoid(h)
    h = jnp.dot(h.astype(jnp.bfloat16), w2_ref[...],
                preferred_element_type=jnp.float32) + b2_ref[...]
    ep = e + _ln(h, s_ref[...], b_ref[...])
    lw = 128
    for k in range(4):
        out_ref[k] = ep[:, k * lw:(k + 1) * lw]


def _tc_edge(e, g, w1e, w2, b2, sc, bi):
    ne, h = e.shape
    te = next(x for x in (1600, 1280, 800, 640, 400, 160) if ne % x == 0)
    rows = pl.BlockSpec((te, h), lambda i: (i, 0))
    full = pl.BlockSpec((h, h), lambda i: (0, 0))
    vec = pl.BlockSpec((1, h), lambda i: (0, 0))
    blocked = pl.BlockSpec((4, te, 128), lambda i: (0, i, 0))
    return pl.pallas_call(
        _edge_body,
        grid=(ne // te,),
        in_specs=[rows, rows, full, full, vec, vec, vec],
        out_specs=blocked,
        out_shape=jax.ShapeDtypeStruct((4, ne, 128), jnp.float32),
    )(e, g, w1e.astype(jnp.bfloat16), w2.astype(jnp.bfloat16),
      b2.reshape(1, h), sc.reshape(1, h), bi.reshape(1, h))


def _node_body(n_ref, agg_ref, w1a_ref, w1b_ref, b1_ref, w2_ref,
               b2_ref, s_ref, b_ref, out_ref):
    nd = n_ref[...]
    h = jnp.dot(nd, w1a_ref[...], preferred_element_type=jnp.float32)
    h = h + jnp.dot(agg_ref[...], w1b_ref[...],
                    preferred_element_type=jnp.float32) + b1_ref[...]
    h = h * jax.nn.sigmoid(h)
    h = jnp.dot(h, w2_ref[...], preferred_element_type=jnp.float32) + b2_ref[...]
    out_ref[...] = nd + _ln(h, s_ref[...], b_ref[...])


def _tc_node(node_f, agg, w1a, w1b, b1, w2, b2, sc, bi):
    n, h = node_f.shape
    tn = 1000
    rows = pl.BlockSpec((tn, h), lambda i: (i, 0))
    full = pl.BlockSpec((h, h), lambda i: (0, 0))
    vec = pl.BlockSpec((1, h), lambda i: (0, 0))
    return pl.pallas_call(
        _node_body,
        grid=(n // tn,),
        in_specs=[rows, rows, full, full, vec, full, vec, vec, vec],
        out_specs=rows,
        out_shape=jax.ShapeDtypeStruct((n, h), jnp.float32),
    )(node_f, agg, w1a, w1b, b1.reshape(1, h), w2, b2.reshape(1, h),
      sc.reshape(1, h), bi.reshape(1, h))


def _sc_gather(psrc, pdst, src_idx, dst_idx):
    n, h = psrc.shape
    e = src_idx.shape[0]
    nw = 32
    per_w = e // nw
    c = 40
    nch = per_w // c
    src3 = src_idx.reshape(nw, nch, c)
    dst3 = dst_idx.reshape(nw, nch, c)
    mesh = plsc.VectorSubcoreMesh(core_axis_name="c", subcore_axis_name="s")

    bufty = pltpu.VMEM((c, h), jnp.float32)

    @functools.partial(
        pl.kernel, mesh=mesh,
        out_type=jax.ShapeDtypeStruct((e, h), jnp.float32),
        scratch_types=[
            pltpu.VMEM((nch, c), jnp.int32), pltpu.VMEM((nch, c), jnp.int32),
            bufty, bufty, bufty, bufty,
            pltpu.SemaphoreType.DMA, pltpu.SemaphoreType.DMA,
            pltpu.SemaphoreType.DMA, pltpu.SemaphoreType.DMA,
        ])
    def k(ps_hbm, pd_hbm, si_hbm, di_hbm, gs_hbm,
          idxs, idxd, bufa0, bufa1, bufb0, bufb1, semg0, semg1, semw0, semw1):
        wid = lax.axis_index("s") * 2 + lax.axis_index("c")
        pltpu.sync_copy(si_hbm.at[wid], idxs)
        pltpu.sync_copy(di_hbm.at[wid], idxd)
        bufs = ((bufa0, bufb0, semg0, semw0), (bufa1, bufb1, semg1, semw1))

        def gath(j, p):
            ba, bb, sg, _ = bufs[p]
            pltpu.async_copy(ps_hbm.at[idxs.at[j]], ba, sg)
            pltpu.async_copy(pd_hbm.at[idxd.at[j]], bb, sg)

        def waitg(p):
            ba, bb, sg, _ = bufs[p]
            pltpu.make_async_copy(ps_hbm.at[idxs.at[0]], ba, sg).wait()
            pltpu.make_async_copy(pd_hbm.at[idxd.at[0]], bb, sg).wait()

        def addpair(p):
            ba, bb, _, _ = bufs[p]

            def arow(r, carry):
                for kk in range(h // 16):
                    sl = pl.ds(kk * 16, 16)
                    ba[r, sl] = ba[r, sl] + bb[r, sl]
                return carry

            lax.fori_loop(0, c, arow, 0)

        def wrote(j, p):
            ba, _, _, sw = bufs[p]
            base = wid * per_w + j * c
            pltpu.async_copy(ba, gs_hbm.at[pl.ds(base, c)], sw)

        def waitw(p):
            ba, _, _, sw = bufs[p]
            pltpu.make_async_copy(ba, gs_hbm.at[pl.ds(0, c)], sw).wait()

        gath(0, 0)
        gath(1, 1)

        def body(i, carry):
            j0 = 2 * i
            waitg(0)
            addpair(0)
            wrote(j0, 0)
            waitg(1)
            addpair(1)
            wrote(j0 + 1, 1)
            waitw(0)

            @pl.when(j0 + 2 < nch)
            def _():
                gath(j0 + 2, 0)

            waitw(1)

            @pl.when(j0 + 3 < nch)
            def _():
                gath(j0 + 3, 1)

            return carry

        lax.fori_loop(0, nch // 2, body, 0)
        if nch % 2:
            waitg(0)
            addpair(0)
            wrote(nch - 1, 0)
            waitw(0)

    return k(psrc, pdst, src3, dst3)


def _sc_scatter(eprime, dst_idx, zrows):
    nblk4, e, cb = eprime.shape
    n = zrows.shape[0]
    nblk = nblk4 // 2
    per_tile = e // 16
    cs = 80
    nch = per_tile // cs
    rc = 400
    nrc = n // rc
    nrit = (nrc + 15) // 16
    dst4 = dst_idx.reshape(16, nch, 1, cs)
    mesh = plsc.VectorSubcoreMesh(core_axis_name="c", subcore_axis_name="s")

    @functools.partial(
        pl.kernel, mesh=mesh,
        out_type=jax.ShapeDtypeStruct((n, nblk4 * cb), jnp.float32),
        scratch_types=[
            pltpu.VMEM((nch, 1, cs), jnp.int32),
            pltpu.VMEM((cs, cb), jnp.float32),
            pltpu.VMEM((cs, cb), jnp.float32),
            pltpu.VMEM_SHARED((n, cb), jnp.float32),
            pltpu.SemaphoreType.DMA, pltpu.SemaphoreType.DMA,
        ])
    def k(ep_hbm, di_hbm, z_hbm, agg_hbm, idxall, ebuf0, ebuf1, acc,
          sem0, sem1):
        c = lax.axis_index("c")
        s = lax.axis_index("s")
        pltpu.sync_copy(di_hbm.at[s], idxall)
        bufs = ((ebuf0, sem0), (ebuf1, sem1))

        for bb in range(nblk):
            blk = bb * 2 + c
            col = blk * cb

            def zbody(j, carry):
                idx = j * 16 + s

                @pl.when(idx < nrc)
                def _():
                    rb = idx * rc
                    pltpu.sync_copy(z_hbm.at[pl.ds(rb, rc)],
                                    acc.at[pl.ds(rb, rc)])
                return carry

            lax.fori_loop(0, nrit, zbody, 0)
            plsc.subcore_barrier()

            def load(j, p):
                buf, sem = bufs[p]
                pltpu.async_copy(
                    ep_hbm.at[blk, pl.ds(s * per_tile + j * cs, cs)],
                    buf, sem)

            def waitl(p):
                buf, sem = bufs[p]
                pltpu.make_async_copy(
                    ep_hbm.at[blk, pl.ds(0, cs)], buf, sem).wait()

            def scat(j, p):
                buf, _ = bufs[p]
                pltpu.sync_copy(buf, acc.at[idxall.at[j, 0]], add=True)

            load(0, 0)

            def body(i, carry):
                j0 = 2 * i
                load(j0 + 1, 1)
                waitl(0)
                scat(j0, 0)
                load(j0 + 2, 0)
                waitl(1)
                scat(j0 + 1, 1)
                return carry

            npairs = (nch - 1) // 2
            lax.fori_loop(0, npairs, body, 0)
            waitl(0)
            if nch % 2 == 0:
                load(nch - 1, 1)
            scat(2 * npairs, 0)
            if nch % 2 == 0:
                waitl(1)
                scat(nch - 1, 1)
            plsc.subcore_barrier()

            def wbody(j, carry):
                idx = j * 16 + s

                @pl.when(idx < nrc)
                def _():
                    rb = idx * rc
                    pltpu.sync_copy(acc.at[pl.ds(rb, rc)],
                                    agg_hbm.at[pl.ds(rb, rc), pl.ds(col, cb)])
                return carry

            lax.fori_loop(0, nrit, wbody, 0)
            plsc.subcore_barrier()

    assert nch >= 2 and nch * cs == per_tile, (nch, per_tile)
    return k(eprime, dst4, zrows)


def kernel(mesh2grid_edge_features, grid_node_features, mesh_node_features,
           edge_index, We1, be1, We2, be2, ge_scale, ge_bias,
           Wn1, bn1, Wn2, bn2, gn_scale, gn_bias):
    h = mesh2grid_edge_features.shape[1]
    n = grid_node_features.shape[0]
    dst = edge_index[:, 0].astype(jnp.int32)
    src = edge_index[:, 1].astype(jnp.int32)

    w1e, w1s, w1d = We1[:h], We1[h:2 * h], We1[2 * h:]
    ps, pd = _tc_preproj(mesh_node_features, grid_node_features, w1s, w1d, be1)
    zrows = jnp.zeros((n, 128), jnp.float32)
    gsum = _sc_gather(ps, pd, src, dst)
    ep = _tc_edge(mesh2grid_edge_features, gsum, w1e, We2, be2,
                  ge_scale, ge_bias)
    agg = _sc_scatter(ep, dst, zrows)
    return _tc_node(grid_node_features, agg, Wn1[:h], Wn1[h:],
                    bn1, Wn2, bn2, gn_scale, gn_bias)
